# baseline (speedup 1.0000x reference)
import functools

import jax
import jax.numpy as jnp
from jax import lax
from jax.experimental import pallas as pl
from jax.experimental.pallas import tpu as pltpu
from jax.experimental.pallas import tpu_sc as plsc

D = 64
B = 16384
S = 20
B3 = B * S
NC = 2
NS = 16
NW = NC * NS
PW1 = B // NW
PW3 = B3 // NW
C = 512
NCH = PW3 // C

_mesh = plsc.VectorSubcoreMesh(core_axis_name="c", subcore_axis_name="s")


@functools.partial(
    pl.kernel,
    mesh=_mesh,
    compiler_params=pltpu.CompilerParams(use_tc_tiling_on_sc=False),
    out_type=[
        jax.ShapeDtypeStruct((B, D), jnp.float32),
        jax.ShapeDtypeStruct((B, D), jnp.float32),
        jax.ShapeDtypeStruct((B3, D), jnp.float32),
    ],
    scratch_types=[
        pltpu.VMEM((C,), jnp.int32),
        pltpu.VMEM((C, D), jnp.float32),
        pltpu.SemaphoreType.DMA,
    ],
)
def _sc_gather(iw_hbm, ow_hbm, nz_hbm, ine_hbm, oute_hbm,
               out1, out2, out3, idx_v, rows_v, sem):
    wid = lax.axis_index("s") * NC + lax.axis_index("c")

    base = wid * PW1
    pltpu.sync_copy(iw_hbm.at[pl.ds(base, C)], idx_v)
    pltpu.async_copy(ine_hbm.at[idx_v], rows_v, sem).wait()
    pltpu.sync_copy(rows_v, out1.at[pl.ds(base, C)])

    pltpu.sync_copy(ow_hbm.at[pl.ds(base, C)], idx_v)
    pltpu.async_copy(oute_hbm.at[idx_v], rows_v, sem).wait()
    pltpu.sync_copy(rows_v, out2.at[pl.ds(base, C)])

    base3 = wid * PW3

    def body(i, carry):
        b = base3 + i * C
        pltpu.sync_copy(nz_hbm.at[pl.ds(b, C)], idx_v)
        pltpu.async_copy(oute_hbm.at[idx_v], rows_v, sem).wait()
        pltpu.sync_copy(rows_v, out3.at[pl.ds(b, C)])
        return carry

    lax.fori_loop(0, NCH, body, 0)


def kernel(input_words, output_words, noise_words, in_embed_w, out_embed_w):
    iw = input_words.astype(jnp.int32)
    ow = output_words.astype(jnp.int32)
    nz = noise_words.astype(jnp.int32)
    out1, out2, out3 = _sc_gather(iw, ow, nz, in_embed_w, out_embed_w)
    return out1, out2, out3.reshape(B, S, D)


# speedup vs baseline: 1.0168x; 1.0168x over previous
import functools

import jax
import jax.numpy as jnp
from jax import lax
from jax.experimental import pallas as pl
from jax.experimental.pallas import tpu as pltpu
from jax.experimental.pallas import tpu_sc as plsc

D = 64
B = 16384
S = 20
B3 = B * S
NC = 2
NS = 16
NW = NC * NS
PW1 = B // NW
PW3 = B3 // NW
C = 512
NCH = PW3 // C
NBUF = 3

_mesh = plsc.VectorSubcoreMesh(core_axis_name="c", subcore_axis_name="s")


@functools.partial(
    pl.kernel,
    mesh=_mesh,
    compiler_params=pltpu.CompilerParams(use_tc_tiling_on_sc=False),
    out_type=[
        jax.ShapeDtypeStruct((B, D), jnp.float32),
        jax.ShapeDtypeStruct((B, D), jnp.float32),
        jax.ShapeDtypeStruct((B3, D), jnp.float32),
    ],
    scratch_types=[
        pltpu.VMEM((PW1,), jnp.int32),
        pltpu.VMEM((PW1,), jnp.int32),
        pltpu.VMEM((PW3,), jnp.int32),
        pltpu.VMEM((NBUF, C, D), jnp.float32),
        pltpu.SemaphoreType.DMA,
        pltpu.SemaphoreType.DMA,
        pltpu.SemaphoreType.DMA,
    ],
)
def _sc_gather(iw_hbm, ow_hbm, nz_hbm, ine_hbm, oute_hbm,
               out1, out2, out3, idx1_v, idx2_v, nidx_v, rows_v,
               sem0, sem1, sem2):
    sems = (sem0, sem1, sem2)
    wid = lax.axis_index("s") * NC + lax.axis_index("c")

    base = wid * PW1
    base3 = wid * PW3

    pltpu.sync_copy(iw_hbm.at[pl.ds(base, PW1)], idx1_v)
    g1 = pltpu.async_copy(ine_hbm.at[idx1_v], rows_v.at[0], sem0)
    pltpu.sync_copy(ow_hbm.at[pl.ds(base, PW1)], idx2_v)
    g2 = pltpu.async_copy(oute_hbm.at[idx2_v], rows_v.at[1], sem1)
    pltpu.sync_copy(nz_hbm.at[pl.ds(base3, PW3)], nidx_v)

    g1.wait()
    pltpu.sync_copy(rows_v.at[0], out1.at[pl.ds(base, PW1)])
    starts = [
        pltpu.async_copy(oute_hbm.at[nidx_v.at[pl.ds(0 * C, C)]],
                         rows_v.at[0], sem0)
    ]
    g2.wait()
    pltpu.sync_copy(rows_v.at[1], out2.at[pl.ds(base, PW1)])
    starts.append(
        pltpu.async_copy(oute_hbm.at[nidx_v.at[pl.ds(1 * C, C)]],
                         rows_v.at[1], sem1)
    )
    starts.append(
        pltpu.async_copy(oute_hbm.at[nidx_v.at[pl.ds(2 * C, C)]],
                         rows_v.at[2], sem2)
    )

    for i in range(NCH):
        b = i % NBUF
        starts[i].wait()
        pltpu.sync_copy(rows_v.at[b], out3.at[pl.ds(base3 + i * C, C)])
        if i + NBUF < NCH:
            starts.append(
                pltpu.async_copy(oute_hbm.at[nidx_v.at[pl.ds((i + NBUF) * C, C)]],
                                 rows_v.at[b], sems[b])
            )


def kernel(input_words, output_words, noise_words, in_embed_w, out_embed_w):
    iw = input_words.astype(jnp.int32)
    ow = output_words.astype(jnp.int32)
    nz = noise_words.astype(jnp.int32)
    out1, out2, out3 = _sc_gather(iw, ow, nz, in_embed_w, out_embed_w)
    return out1, out2, out3.reshape(B, S, D)


# speedup vs baseline: 1.2858x; 1.2646x over previous
import functools

import jax
import jax.numpy as jnp
from jax import lax
from jax.experimental import pallas as pl
from jax.experimental.pallas import tpu as pltpu
from jax.experimental.pallas import tpu_sc as plsc

D = 64
W = 128
B = 16384
S = 20
B3 = B * S
NC = 2
NS = 16
NW = NC * NS
PW1 = B // NW
PW3 = B3 // NW
C = 256
NBUF = 3
UNROLL = 8

_mesh = plsc.VectorSubcoreMesh(core_axis_name="c", subcore_axis_name="s")


@functools.partial(
    pl.kernel,
    mesh=_mesh,
    out_type=[
        jax.ShapeDtypeStruct((B, W), jnp.float32),
        jax.ShapeDtypeStruct((B, W), jnp.float32),
        jax.ShapeDtypeStruct((B3, W), jnp.float32),
    ],
    scratch_types=[
        pltpu.VMEM((C,), jnp.int32),
        pltpu.VMEM((NBUF, C, W), jnp.float32),
        pltpu.SemaphoreType.DMA,
        pltpu.SemaphoreType.DMA,
        pltpu.SemaphoreType.DMA,
    ],
)
def _sc_gather(iw_hbm, ow_hbm, nz_hbm, ine_hbm, oute_hbm,
               out1, out2, out3, idx_v, rows_v, sem0, sem1, sem2):
    sems = (sem0, sem1, sem2)
    wid = lax.axis_index("s") * NC + lax.axis_index("c")

    base = wid * PW1
    base3 = wid * PW3

    def fire_chunk(src_idx_hbm, start, table, b):
        pltpu.sync_copy(src_idx_hbm.at[pl.ds(start, C)], idx_v)

        def body(k, carry):
            iv = idx_v[pl.ds(k * 16, 16)]
            for u in range(16):
                i = k * 16 + u
                pltpu.async_copy(
                    table.at[iv[u]], rows_v.at[b, i, pl.ds(0, D)], sems[b]
                )
            return carry

        lax.fori_loop(0, C // 16, body, 0)

    def drain_chunk(table, b):
        def body(k, carry):
            for u in range(UNROLL):
                i = k * UNROLL + u
                pltpu.make_async_copy(
                    table.at[0], rows_v.at[b, i, pl.ds(0, D)], sems[b]
                ).wait()
            return carry

        lax.fori_loop(0, C // UNROLL, body, 0)

    chunks = []
    for i in range(PW1 // C):
        chunks.append((iw_hbm, base + i * C, ine_hbm, out1, base + i * C))
    for i in range(PW1 // C):
        chunks.append((ow_hbm, base + i * C, oute_hbm, out2, base + i * C))

    n = len(chunks)
    for j in range(min(2, n)):
        fire_chunk(chunks[j][0], chunks[j][1], chunks[j][2], j % 2)
    for j in range(n):
        b = j % 2
        drain_chunk(chunks[j][2], b)
        pltpu.sync_copy(rows_v.at[b], chunks[j][3].at[pl.ds(chunks[j][4], C)])
        if j + 2 < n:
            nxt = chunks[j + 2]
            fire_chunk(nxt[0], nxt[1], nxt[2], b)

    NCH3 = PW3 // C
    fire_chunk(nz_hbm, base3 + 0 * C, oute_hbm, 0)
    fire_chunk(nz_hbm, base3 + 1 * C, oute_hbm, 1)

    def pair(t, carry):
        for b in range(2):
            j = t * 2 + b
            drain_chunk(oute_hbm, b)
            pltpu.sync_copy(rows_v.at[b], out3.at[pl.ds(base3 + j * C, C)])
            fire_chunk(nz_hbm, base3 + (j + 2) * C, oute_hbm, b)
        return carry

    lax.fori_loop(0, NCH3 // 2 - 1, pair, 0)
    for b in range(2):
        j = NCH3 - 2 + b
        drain_chunk(oute_hbm, b)
        pltpu.sync_copy(rows_v.at[b], out3.at[pl.ds(base3 + j * C, C)])


def kernel(input_words, output_words, noise_words, in_embed_w, out_embed_w):
    iw = input_words.astype(jnp.int32)
    ow = output_words.astype(jnp.int32)
    nz = noise_words.astype(jnp.int32)
    out1, out2, out3 = _sc_gather(iw, ow, nz, in_embed_w, out_embed_w)
    return (
        out1[:, :D],
        out2[:, :D],
        out3[:, :D].reshape(B, S, D),
    )


# speedup vs baseline: 1.4821x; 1.1527x over previous
import functools

import jax
import jax.numpy as jnp
from jax import lax
from jax.experimental import pallas as pl
from jax.experimental.pallas import tpu as pltpu
from jax.experimental.pallas import tpu_sc as plsc

D = 64
W = 128
B = 16384
S = 20
B3 = B * S
NC = 2
NS = 16
NW = NC * NS
PW1 = B // NW
PW3 = B3 // NW
C = 256
NBUF = 2

_mesh = plsc.VectorSubcoreMesh(core_axis_name="c", subcore_axis_name="s")


def _worker_id():
    return lax.axis_index("s") * NC + lax.axis_index("c")


@functools.partial(
    pl.kernel,
    mesh=_mesh,
    out_type=jax.ShapeDtypeStruct((B * D // W, W), jnp.float32),
    scratch_types=[
        pltpu.VMEM((PW1,), jnp.int32),
        pltpu.VMEM((1, PW1 // 2, W), jnp.float32),
        pltpu.SemaphoreType.DMA,
    ],
)
def _sc_gather_in(iw_hbm, ine_hbm, out1, idx_v, rows_v, sem):
    base = _worker_id() * PW1
    pltpu.sync_copy(iw_hbm.at[pl.ds(base, PW1)], idx_v)

    def fire(k, carry):
        iv = idx_v[pl.ds(k * 16, 16)]
        for u in range(16):
            i = k * 16 + u
            pltpu.async_copy(
                ine_hbm.at[iv[u]],
                rows_v.at[0, i // 2, pl.ds((i % 2) * D, D)],
                sem,
            )
        return carry

    lax.fori_loop(0, PW1 // 16, fire, 0)

    def drain(k, carry):
        for u in range(16):
            i = k * 16 + u
            pltpu.make_async_copy(
                ine_hbm.at[0], rows_v.at[0, i // 2, pl.ds((i % 2) * D, D)], sem
            ).wait()
        return carry

    lax.fori_loop(0, PW1 // 16, drain, 0)
    pltpu.sync_copy(rows_v.at[0], out1.at[pl.ds(pl.multiple_of(base // 2, 8), PW1 // 2)])


@functools.partial(
    pl.kernel,
    mesh=_mesh,
    out_type=[
        jax.ShapeDtypeStruct((B * D // W, W), jnp.float32),
        jax.ShapeDtypeStruct((B3 * D // W, W), jnp.float32),
    ],
    scratch_types=[
        pltpu.VMEM((C,), jnp.int32),
        pltpu.VMEM((NBUF, C // 2, W), jnp.float32),
        pltpu.SemaphoreType.DMA,
        pltpu.SemaphoreType.DMA,
    ],
)
def _sc_gather_out(ow_hbm, nz_hbm, oute_hbm, out2, out3, idx_v, rows_v,
                   sem0, sem1):
    sems = (sem0, sem1)
    wid = _worker_id()
    base = wid * PW1
    base3 = wid * PW3

    def fire_chunk(src_idx_hbm, start, b):
        pltpu.sync_copy(src_idx_hbm.at[pl.ds(start, C)], idx_v)

        def body(k, carry):
            iv = idx_v[pl.ds(k * 16, 16)]
            for u in range(16):
                i = k * 16 + u
                pltpu.async_copy(
                    oute_hbm.at[iv[u]],
                    rows_v.at[b, i // 2, pl.ds((i % 2) * D, D)],
                    sems[b],
                )
            return carry

        lax.fori_loop(0, C // 16, body, 0)

    def drain_chunk(b):
        def body(k, carry):
            for u in range(16):
                i = k * 16 + u
                pltpu.make_async_copy(
                    oute_hbm.at[0],
                    rows_v.at[b, i // 2, pl.ds((i % 2) * D, D)],
                    sems[b],
                ).wait()
            return carry

        lax.fori_loop(0, C // 16, body, 0)

    def store_chunk(out, row_off, b):
        pltpu.sync_copy(rows_v.at[b], out.at[pl.ds(pl.multiple_of(row_off // 2, 8), C // 2)])

    fire_chunk(ow_hbm, base, 0)
    fire_chunk(ow_hbm, base + C, 1)
    drain_chunk(0)
    store_chunk(out2, base, 0)
    fire_chunk(nz_hbm, base3, 0)
    drain_chunk(1)
    store_chunk(out2, base + C, 1)
    fire_chunk(nz_hbm, base3 + C, 1)

    NCH3 = PW3 // C

    def pair(t, carry):
        for b in range(2):
            j = t * 2 + b
            drain_chunk(b)
            store_chunk(out3, base3 + j * C, b)
            fire_chunk(nz_hbm, base3 + (j + 2) * C, b)
        return carry

    lax.fori_loop(0, NCH3 // 2 - 1, pair, 0)
    for b in range(2):
        j = NCH3 - 2 + b
        drain_chunk(b)
        store_chunk(out3, base3 + j * C, b)


def kernel(input_words, output_words, noise_words, in_embed_w, out_embed_w):
    iw = input_words.astype(jnp.int32)
    ow = output_words.astype(jnp.int32)
    nz = noise_words.astype(jnp.int32)
    out1 = _sc_gather_in(iw, in_embed_w)
    out2, out3 = _sc_gather_out(ow, nz, out_embed_w)
    return (
        out1.reshape(B, D),
        out2.reshape(B, D),
        out3.reshape(B, S, D),
    )


# speedup vs baseline: 1.7221x; 1.1619x over previous
import functools

import jax
import jax.numpy as jnp
from jax import lax
from jax.experimental import pallas as pl
from jax.experimental.pallas import tpu as pltpu
from jax.experimental.pallas import tpu_sc as plsc

D = 64
W = 128
B = 16384
S = 20
B3 = B * S
NC = 2
NS = 16
NW = NC * NS
PW1 = B // NW
PW3 = B3 // NW
C = 256
CB = 16
PWB = B // NW
NBUF = 2

_mesh = plsc.VectorSubcoreMesh(core_axis_name="c", subcore_axis_name="s")


def _worker_id():
    return lax.axis_index("s") * NC + lax.axis_index("c")


@functools.partial(
    pl.kernel,
    mesh=_mesh,
    out_type=jax.ShapeDtypeStruct((B * D // W, W), jnp.float32),
    scratch_types=[
        pltpu.VMEM((PW1,), jnp.int32),
        pltpu.VMEM((1, PW1 // 2, W), jnp.float32),
        pltpu.SemaphoreType.DMA,
    ],
)
def _sc_gather_in(iw_hbm, ine_hbm, out1, idx_v, rows_v, sem):
    base = _worker_id() * PW1
    pltpu.sync_copy(iw_hbm.at[pl.ds(base, PW1)], idx_v)

    def fire(k, carry):
        iv = idx_v[pl.ds(k * 16, 16)]
        for u in range(16):
            i = k * 16 + u
            pltpu.async_copy(
                ine_hbm.at[iv[u]],
                rows_v.at[0, i // 2, pl.ds((i % 2) * D, D)],
                sem,
            )
        return carry

    lax.fori_loop(0, PW1 // 16, fire, 0)

    def drain(k, carry):
        for u in range(16):
            i = k * 16 + u
            pltpu.make_async_copy(
                ine_hbm.at[0], rows_v.at[0, i // 2, pl.ds((i % 2) * D, D)], sem
            ).wait()
        return carry

    lax.fori_loop(0, PW1 // 16, drain, 0)
    pltpu.sync_copy(rows_v.at[0], out1.at[pl.ds(pl.multiple_of(base // 2, 8), PW1 // 2)])


@functools.partial(
    pl.kernel,
    mesh=_mesh,
    out_type=[
        jax.ShapeDtypeStruct((B * D // W, W), jnp.float32),
        jax.ShapeDtypeStruct((B, S * D), jnp.float32),
    ],
    scratch_types=[
        pltpu.VMEM((CB * S,), jnp.int32),
        pltpu.VMEM((NBUF, C // 2, W), jnp.float32),
        pltpu.VMEM((NBUF, CB, S * D), jnp.float32),
        pltpu.SemaphoreType.DMA,
        pltpu.SemaphoreType.DMA,
    ],
)
def _sc_gather_out(ow_hbm, nz_hbm, oute_hbm, out2, out3, idx_v, rows_v,
                   rows3_v, sem0, sem1):
    sems = (sem0, sem1)
    wid = _worker_id()
    base = wid * PW1
    base3 = wid * PW3
    baseb = wid * PWB

    def fire_chunk(src_idx_hbm, start, b):
        pltpu.sync_copy(src_idx_hbm.at[pl.ds(start, C)], idx_v.at[pl.ds(0, C)])

        def body(k, carry):
            iv = idx_v[pl.ds(k * 16, 16)]
            for u in range(16):
                i = k * 16 + u
                pltpu.async_copy(
                    oute_hbm.at[iv[u]],
                    rows_v.at[b, i // 2, pl.ds((i % 2) * D, D)],
                    sems[b],
                )
            return carry

        lax.fori_loop(0, C // 16, body, 0)

    def drain_chunk(b):
        def body(k, carry):
            for u in range(16):
                i = k * 16 + u
                pltpu.make_async_copy(
                    oute_hbm.at[0],
                    rows_v.at[b, i // 2, pl.ds((i % 2) * D, D)],
                    sems[b],
                ).wait()
            return carry

        lax.fori_loop(0, C // 16, body, 0)

    def store_chunk(out, row_off, b):
        pltpu.sync_copy(rows_v.at[b], out.at[pl.ds(pl.multiple_of(row_off // 2, 8), C // 2)])

    def fire_chunk3(start_row, b):
        pltpu.sync_copy(nz_hbm.at[pl.ds(start_row, CB * S)], idx_v)

        def body(k, carry):
            iv = idx_v[pl.ds(k * 16, 16)]
            for u in range(16):
                i = k * 16 + u
                pltpu.async_copy(
                    oute_hbm.at[iv[u]],
                    rows3_v.at[b, i // S, pl.ds((i % S) * D, D)],
                    sems[b],
                )
            return carry

        lax.fori_loop(0, (CB * S) // 16, body, 0)

    def drain_chunk3(b):
        def body(k, carry):
            for u in range(16):
                i = k * 16 + u
                pltpu.make_async_copy(
                    oute_hbm.at[0],
                    rows3_v.at[b, i // S, pl.ds((i % S) * D, D)],
                    sems[b],
                ).wait()
            return carry

        lax.fori_loop(0, (CB * S) // 16, body, 0)

    def store_chunk3(brow, b):
        pltpu.sync_copy(
            rows3_v.at[b], out3.at[pl.ds(pl.multiple_of(brow, 8), CB)]
        )

    fire_chunk(ow_hbm, base, 0)
    fire_chunk(ow_hbm, base + C, 1)
    drain_chunk(0)
    store_chunk(out2, base, 0)
    fire_chunk3(base3, 0)
    drain_chunk(1)
    store_chunk(out2, base + C, 1)
    fire_chunk3(base3 + CB * S, 1)

    NCH3 = PWB // CB

    def pair(t, carry):
        for b in range(2):
            j = t * 2 + b
            drain_chunk3(b)
            store_chunk3(baseb + j * CB, b)
            fire_chunk3(base3 + (j + 2) * CB * S, b)
        return carry

    lax.fori_loop(0, NCH3 // 2 - 1, pair, 0)
    for b in range(2):
        j = NCH3 - 2 + b
        drain_chunk3(b)
        store_chunk3(baseb + j * CB, b)


def kernel(input_words, output_words, noise_words, in_embed_w, out_embed_w):
    iw = input_words.astype(jnp.int32)
    ow = output_words.astype(jnp.int32)
    nz = noise_words.astype(jnp.int32)
    out1 = _sc_gather_in(iw, in_embed_w)
    out2, out3 = _sc_gather_out(ow, nz, out_embed_w)
    return (
        out1.reshape(B, D),
        out2.reshape(B, D),
        out3.reshape(B, S, D),
    )
